# baseline (speedup 1.0000x reference)
import functools

import jax
import jax.numpy as jnp
from jax import lax
from jax.experimental import pallas as pl
from jax.experimental.pallas import tpu as pltpu
from jax.experimental.pallas import tpu_sc as plsc

_D = 64
_HP = 56
_DP = 128
_S = 3


@functools.lru_cache(maxsize=None)
def _build_gather(B0, H):
    info = plsc.get_sparse_core_info()
    NC, NS = info.num_cores, info.num_subcores
    NW = NC * NS
    assert B0 % NW == 0 and H > 2 * _S
    bw = B0 // NW
    mesh = plsc.VectorSubcoreMesh(core_axis_name="c", subcore_axis_name="s")

    T = (H - 1) // _S

    @functools.partial(
        pl.kernel,
        mesh=mesh,
        out_type=jax.ShapeDtypeStruct((B0, _HP, _DP), jnp.float32),
        scratch_types=[
            pltpu.VMEM((H, bw), jnp.int32),
            pltpu.VMEM((_S, bw, _D), jnp.float32),
            pltpu.SemaphoreType.DMA,
            pltpu.SemaphoreType.DMA,
            pltpu.SemaphoreType.DMA,
            pltpu.SemaphoreType.DMA,
            pltpu.SemaphoreType.DMA,
            pltpu.SemaphoreType.DMA,
        ],
        compiler_params=pltpu.CompilerParams(use_tc_tiling_on_sc=False),
    )
    def gather_kernel(idxt_hbm, table_hbm, out_hbm, idx_v, rows_v,
                      g0, g1, g2, o0, o1, o2):
        gsem = (g0, g1, g2)
        osem = (o0, o1, o2)
        wid = lax.axis_index("s") * NC + lax.axis_index("c")
        b0 = wid * bw

        def gather_desc(j, h):
            return pltpu.make_async_copy(
                table_hbm.at[idx_v.at[h]], rows_v.at[j], gsem[j])

        def out_desc(j, h):
            return pltpu.make_async_copy(
                rows_v.at[j],
                out_hbm.at[pl.ds(b0, bw), h, pl.ds(0, _D)],
                osem[j])

        def visit(j, h, wait_prev_write, prefetch):
            if prefetch:
                jn = (j + 1) % _S
                if wait_prev_write:
                    out_desc(jn, h + 1 - _S).wait()
                gather_desc(jn, h + 1).start()
            gather_desc(j, h).wait()
            out_desc(j, h).start()

        pltpu.sync_copy(idxt_hbm.at[pl.ds(0, H), pl.ds(b0, bw)], idx_v)
        gather_desc(0, 0).start()

        for h in range(_S):
            visit(h % _S, h, wait_prev_write=(h + 1 - _S >= 0), prefetch=True)

        def body(tt, carry):
            for j in range(_S):
                h = tt * _S + j
                visit(j, h, wait_prev_write=True, prefetch=True)
            return carry

        lax.fori_loop(1, T, body, 0, unroll=False)

        for h in range(T * _S, H):
            visit(h % _S, h, wait_prev_write=True, prefetch=(h + 1 < H))

        for h in range(H - _S, H):
            out_desc(h % _S, h).wait()

    return gather_kernel


def kernel(indices, table):
    B0, H = indices.shape
    idxt = indices.astype(jnp.int32).T
    padded = _build_gather(B0, H)(idxt, table)
    return padded[:, :H, :_D]


# speedup vs baseline: 1.0074x; 1.0074x over previous
import functools

import jax
import jax.numpy as jnp
from jax import lax
from jax.experimental import pallas as pl
from jax.experimental.pallas import tpu as pltpu
from jax.experimental.pallas import tpu_sc as plsc

_D = 64
_HP = 56
_DP = 128


@functools.lru_cache(maxsize=None)
def _build_gather(B0, H):
    info = plsc.get_sparse_core_info()
    NC, NS = info.num_cores, info.num_subcores
    NW = NC * NS
    assert B0 % NW == 0 and H % 2 == 0
    bw = B0 // NW
    mesh = plsc.VectorSubcoreMesh(core_axis_name="c", subcore_axis_name="s")

    @functools.partial(
        pl.kernel,
        mesh=mesh,
        out_type=jax.ShapeDtypeStruct((B0, _HP, _DP), jnp.float32),
        scratch_types=[
            pltpu.VMEM((H, bw), jnp.int32),
            pltpu.VMEM((2, bw, _D), jnp.float32),
            pltpu.SemaphoreType.DMA,
            pltpu.SemaphoreType.DMA,
            pltpu.SemaphoreType.DMA,
            pltpu.SemaphoreType.DMA,
        ],
        compiler_params=pltpu.CompilerParams(use_tc_tiling_on_sc=False),
    )
    def gather_kernel(idxt_hbm, table_hbm, out_hbm, idx_v, rows_v,
                      g0, g1, o0, o1):
        gsem = (g0, g1)
        osem = (o0, o1)
        wid = lax.axis_index("s") * NC + lax.axis_index("c")
        b0 = wid * bw

        def gather_desc(b, h):
            return pltpu.make_async_copy(
                table_hbm.at[idx_v.at[h]], rows_v.at[b], gsem[b])

        def out_desc(b, h):
            return pltpu.make_async_copy(
                rows_v.at[b],
                out_hbm.at[pl.ds(b0, bw), h, pl.ds(0, _D)],
                osem[b])

        pltpu.sync_copy(idxt_hbm.at[pl.ds(0, H), pl.ds(b0, bw)], idx_v)
        for b in range(2):
            gather_desc(b, b).start()

        npairs = H // 2

        def body(tt, carry):
            for b in range(2):
                h = tt * 2 + b
                gather_desc(b, h).wait()
                out_desc(b, h).start()
                out_desc(b, h).wait()
                gather_desc(b, h + 2).start()
            return carry

        lax.fori_loop(0, npairs - 1, body, 0)

        for b in range(2):
            h = (npairs - 1) * 2 + b
            gather_desc(b, h).wait()
            out_desc(b, h).start()
        for b in range(2):
            h = (npairs - 1) * 2 + b
            out_desc(b, h).wait()

    return gather_kernel


def kernel(indices, table):
    B0, H = indices.shape
    idxt = indices.astype(jnp.int32).T
    padded = _build_gather(B0, H)(idxt, table)
    return padded[:, :H, :_D]
